# baseline (speedup 1.0000x reference)
import functools

import jax
import jax.numpy as jnp
from jax import lax
from jax.experimental import pallas as pl
from jax.experimental.pallas import tpu as pltpu
from jax.experimental.pallas import tpu_sc as plsc

_NUM_CORES = 2
_NUM_SUBCORES = 16
_LANES = 16
_NUM_WORKERS = _NUM_CORES * _NUM_SUBCORES

_M = 4
_B = 32
_N = 8192
_FLOATS = _N * 3
_CHUNK_PTS = 2048
_CHUNK_F = _CHUNK_PTS * 3
_NCHUNKS = _N // _CHUNK_PTS


def _se3_body(pos_hbm, coef_hbm, out_hbm, in_v, out_v0, out_v1, out_v2, out_v3, coef_v):
    out_v = [out_v0, out_v1, out_v2, out_v3]
    c = lax.axis_index("c")
    s = lax.axis_index("s")
    w = s * _NUM_CORES + c

    for m in range(_M):
        pltpu.sync_copy(coef_hbm.at[m * _B + w], coef_v[m])

    rows = [coef_v[m][...] for m in range(_M)]
    cf = [[rows[m][j] for j in range(12)] for m in range(_M)]

    iota3 = lax.iota(jnp.int32, _LANES) * 3

    def step(i, carry):
        pidx = iota3 + i * (3 * _LANES)
        x = plsc.load_gather(in_v, [pidx])
        y = plsc.load_gather(in_v, [pidx + 1])
        z = plsc.load_gather(in_v, [pidx + 2])
        for m in range(_M):
            r = cf[m]
            ox = x * r[0] + y * r[1] + z * r[2] + r[9]
            oy = x * r[3] + y * r[4] + z * r[5] + r[10]
            oz = x * r[6] + y * r[7] + z * r[8] + r[11]
            plsc.store_scatter(out_v[m], [pidx], ox)
            plsc.store_scatter(out_v[m], [pidx + 1], oy)
            plsc.store_scatter(out_v[m], [pidx + 2], oz)
        return carry

    for k in range(_NCHUNKS):
        pltpu.sync_copy(pos_hbm.at[w, pl.ds(k * _CHUNK_F, _CHUNK_F)], in_v)
        lax.fori_loop(0, _CHUNK_PTS // _LANES, step, 0)
        for m in range(_M):
            pltpu.sync_copy(
                out_v[m],
                out_hbm.at[m * _B + w, pl.ds(k * _CHUNK_F, _CHUNK_F)],
            )


@jax.jit
def kernel(trans, pos):
    m, b = trans.shape[0], trans.shape[1]
    n = pos.shape[1]
    t = trans.reshape(m * b, 4, 4)
    rot = t[:, 0:3, 0:3].reshape(m * b, 9)
    shift = t[:, 0:3, 3]
    coef = jnp.concatenate(
        [rot, shift, jnp.zeros((m * b, 4), jnp.float32)], axis=1
    )
    pos_flat = pos.reshape(b, n * 3)

    mesh = plsc.VectorSubcoreMesh(
        core_axis_name="c", subcore_axis_name="s",
        num_cores=_NUM_CORES, num_subcores=_NUM_SUBCORES,
    )
    out = pl.kernel(
        _se3_body,
        out_type=jax.ShapeDtypeStruct((m * b, n * 3), jnp.float32),
        mesh=mesh,
        scratch_types=[
            pltpu.VMEM((_CHUNK_F,), jnp.float32),
            pltpu.VMEM((_CHUNK_F,), jnp.float32),
            pltpu.VMEM((_CHUNK_F,), jnp.float32),
            pltpu.VMEM((_CHUNK_F,), jnp.float32),
            pltpu.VMEM((_CHUNK_F,), jnp.float32),
            [pltpu.VMEM((16,), jnp.float32) for _ in range(4)],
        ],
        compiler_params=pltpu.CompilerParams(needs_layout_passes=False),
    )(pos_flat, coef)
    return out.reshape(m * b, n, 3)


# speedup vs baseline: 1.0060x; 1.0060x over previous
import functools

import jax
import jax.numpy as jnp
from jax import lax
from jax.experimental import pallas as pl
from jax.experimental.pallas import tpu as pltpu
from jax.experimental.pallas import tpu_sc as plsc

_NUM_CORES = 2
_NUM_SUBCORES = 16
_LANES = 16
_NUM_WORKERS = _NUM_CORES * _NUM_SUBCORES

_M = 4
_B = 32
_N = 8192
_CHUNK_PTS = 1024
_CHUNK_F = _CHUNK_PTS * 3
_NCHUNKS = _N // _CHUNK_PTS
_STEPS = _CHUNK_PTS // _LANES


def _se3_body(pos_hbm, coef_hbm, out_hbm,
              in0, in1, o00, o01, o02, o03, o10, o11, o12, o13,
              coef_v, in_sems, out_sems):
    in_v = [in0, in1]
    out_v = [[o00, o01, o02, o03], [o10, o11, o12, o13]]
    c = lax.axis_index("c")
    s = lax.axis_index("s")
    w = s * _NUM_CORES + c

    for m in range(_M):
        pltpu.sync_copy(coef_hbm.at[m * _B + w], coef_v[m])
    rows = [coef_v[m][...] for m in range(_M)]
    cf = [[rows[m][j] for j in range(12)] for m in range(_M)]

    iota3 = lax.iota(jnp.int32, _LANES) * 3

    def in_copy(k):
        slot = k % 2
        return pltpu.make_async_copy(
            pos_hbm.at[w, pl.ds(k * _CHUNK_F, _CHUNK_F)],
            in_v[slot], in_sems[slot])

    def out_copy(k, m):
        slot = k % 2
        return pltpu.make_async_copy(
            out_v[slot][m],
            out_hbm.at[m * _B + w, pl.ds(k * _CHUNK_F, _CHUNK_F)],
            out_sems[slot])

    def compute(slot):
        src = in_v[slot]
        dsts = out_v[slot]

        @plsc.parallel_loop(0, _STEPS, unroll=4)
        def body(i):
            pidx = iota3 + i * (3 * _LANES)
            x = plsc.load_gather(src, [pidx])
            y = plsc.load_gather(src, [pidx + 1])
            z = plsc.load_gather(src, [pidx + 2])
            for m in range(_M):
                r = cf[m]
                ox = x * r[0] + y * r[1] + z * r[2] + r[9]
                oy = x * r[3] + y * r[4] + z * r[5] + r[10]
                oz = x * r[6] + y * r[7] + z * r[8] + r[11]
                plsc.store_scatter(dsts[m], [pidx], ox)
                plsc.store_scatter(dsts[m], [pidx + 1], oy)
                plsc.store_scatter(dsts[m], [pidx + 2], oz)

    in_copy(0).start()
    for k in range(_NCHUNKS):
        slot = k % 2
        if k + 1 < _NCHUNKS:
            in_copy(k + 1).start()
        in_copy(k).wait()
        if k >= 2:
            for m in range(_M):
                out_copy(k - 2, m).wait()
        compute(slot)
        for m in range(_M):
            out_copy(k, m).start()
    for k in (_NCHUNKS - 2, _NCHUNKS - 1):
        for m in range(_M):
            out_copy(k, m).wait()


@jax.jit
def kernel(trans, pos):
    m, b = trans.shape[0], trans.shape[1]
    n = pos.shape[1]
    t = trans.reshape(m * b, 4, 4)
    rot = t[:, 0:3, 0:3].reshape(m * b, 9)
    shift = t[:, 0:3, 3]
    coef = jnp.concatenate(
        [rot, shift, jnp.zeros((m * b, 4), jnp.float32)], axis=1
    )
    pos_flat = pos.reshape(b, n * 3)

    mesh = plsc.VectorSubcoreMesh(
        core_axis_name="c", subcore_axis_name="s",
        num_cores=_NUM_CORES, num_subcores=_NUM_SUBCORES,
    )
    out = pl.kernel(
        _se3_body,
        out_type=jax.ShapeDtypeStruct((m * b, n * 3), jnp.float32),
        mesh=mesh,
        scratch_types=(
            [pltpu.VMEM((_CHUNK_F,), jnp.float32) for _ in range(10)]
            + [
                [pltpu.VMEM((16,), jnp.float32) for _ in range(_M)],
                [pltpu.SemaphoreType.DMA for _ in range(2)],
                [pltpu.SemaphoreType.DMA for _ in range(2)],
            ]
        ),
        compiler_params=pltpu.CompilerParams(needs_layout_passes=False),
    )(pos_flat, coef)
    return out.reshape(m * b, n, 3)


# speedup vs baseline: 3.3000x; 3.2804x over previous
import functools

import jax
import jax.numpy as jnp
from jax import lax
from jax.experimental import pallas as pl
from jax.experimental.pallas import tpu as pltpu
from jax.experimental.pallas import tpu_sc as plsc

_NUM_CORES = 2
_NUM_SUBCORES = 16
_LANES = 16

_M = 4
_B = 32
_N = 8192
_CHUNK = 2048
_NCHUNKS = _N // _CHUNK
_STEPS = _CHUNK // _LANES


def _se3_body(xyz_hbm, coef_hbm, out_hbm, in_v, out_v, coef_v,
              in_sems, out_sems):
    c = lax.axis_index("c")
    s = lax.axis_index("s")
    w = s * _NUM_CORES + c

    for m in range(_M):
        pltpu.sync_copy(coef_hbm.at[m * _B + w], coef_v[m])
    rows = [coef_v[m][...] for m in range(_M)]
    cf = [[rows[m][j] for j in range(12)] for m in range(_M)]

    def in_copy(k, d):
        slot = k % 2
        return pltpu.make_async_copy(
            xyz_hbm.at[d, w, pl.ds(k * _CHUNK, _CHUNK)],
            in_v[slot][d], in_sems[slot])

    def out_copy(k, m, d):
        slot = k % 2
        return pltpu.make_async_copy(
            out_v[slot][3 * m + d],
            out_hbm.at[d, m * _B + w, pl.ds(k * _CHUNK, _CHUNK)],
            out_sems[slot])

    def compute(slot):
        xs, ys, zs = in_v[slot]
        dsts = out_v[slot]

        @plsc.parallel_loop(0, _STEPS, unroll=4)
        def body(i):
            o = i * _LANES
            sl = pl.ds(o, _LANES)
            x = xs[sl]
            y = ys[sl]
            z = zs[sl]
            for m in range(_M):
                r = cf[m]
                dsts[3 * m][sl] = x * r[0] + y * r[1] + z * r[2] + r[9]
                dsts[3 * m + 1][sl] = x * r[3] + y * r[4] + z * r[5] + r[10]
                dsts[3 * m + 2][sl] = x * r[6] + y * r[7] + z * r[8] + r[11]

    for d in range(3):
        in_copy(0, d).start()
    for k in range(_NCHUNKS):
        slot = k % 2
        if k + 1 < _NCHUNKS:
            for d in range(3):
                in_copy(k + 1, d).start()
        for d in range(3):
            in_copy(k, d).wait()
        if k >= 2:
            for m in range(_M):
                for d in range(3):
                    out_copy(k - 2, m, d).wait()
        compute(slot)
        for m in range(_M):
            for d in range(3):
                out_copy(k, m, d).start()
    for k in (_NCHUNKS - 2, _NCHUNKS - 1):
        for m in range(_M):
            for d in range(3):
                out_copy(k, m, d).wait()


@jax.jit
def kernel(trans, pos):
    m, b = trans.shape[0], trans.shape[1]
    n = pos.shape[1]
    t = trans.reshape(m * b, 4, 4)
    rot = t[:, 0:3, 0:3].reshape(m * b, 9)
    shift = t[:, 0:3, 3]
    coef = jnp.concatenate(
        [rot, shift, jnp.zeros((m * b, 4), jnp.float32)], axis=1
    )
    xyz = jnp.transpose(pos, (2, 0, 1))

    mesh = plsc.VectorSubcoreMesh(
        core_axis_name="c", subcore_axis_name="s",
        num_cores=_NUM_CORES, num_subcores=_NUM_SUBCORES,
    )
    out = pl.kernel(
        _se3_body,
        out_type=jax.ShapeDtypeStruct((3, m * b, n), jnp.float32),
        mesh=mesh,
        scratch_types=(
            [
                [[pltpu.VMEM((_CHUNK,), jnp.float32) for _ in range(3)]
                 for _ in range(2)],
                [[pltpu.VMEM((_CHUNK,), jnp.float32) for _ in range(3 * _M)]
                 for _ in range(2)],
                [pltpu.VMEM((16,), jnp.float32) for _ in range(_M)],
                [pltpu.SemaphoreType.DMA for _ in range(2)],
                [pltpu.SemaphoreType.DMA for _ in range(2)],
            ]
        ),
        compiler_params=pltpu.CompilerParams(needs_layout_passes=False),
    )(xyz, coef)
    return jnp.transpose(out, (1, 2, 0))


# speedup vs baseline: 3.3874x; 1.0265x over previous
import functools

import jax
import jax.numpy as jnp
from jax import lax
from jax.experimental import pallas as pl
from jax.experimental.pallas import tpu as pltpu
from jax.experimental.pallas import tpu_sc as plsc

_NUM_CORES = 2
_NUM_SUBCORES = 16
_LANES = 16

_M = 4
_B = 32
_N = 8192
_CHUNK = 4096
_NCHUNKS = _N // _CHUNK
_STEPS = _CHUNK // _LANES


def _se3_body(xyz_hbm, coef_hbm, out_hbm, in_v, out_v, coef_v,
              in_sems, out_sems):
    c = lax.axis_index("c")
    s = lax.axis_index("s")
    w = s * _NUM_CORES + c

    pltpu.sync_copy(coef_hbm.at[w], coef_v)
    rows = [coef_v[pl.ds(m * 16, 16)] for m in range(_M)]
    cf = [[rows[m][j] for j in range(12)] for m in range(_M)]

    def in_copy(k, d):
        slot = k % 2
        return pltpu.make_async_copy(
            xyz_hbm.at[d, w, pl.ds(k * _CHUNK, _CHUNK)],
            in_v[slot][d], in_sems[slot])

    def out_copy(k, m, d):
        slot = k % 2
        return pltpu.make_async_copy(
            out_v[slot][3 * m + d],
            out_hbm.at[d, m * _B + w, pl.ds(k * _CHUNK, _CHUNK)],
            out_sems[slot])

    def compute(slot):
        xs, ys, zs = in_v[slot]
        dsts = out_v[slot]

        @plsc.parallel_loop(0, _STEPS, unroll=4)
        def body(i):
            o = i * _LANES
            sl = pl.ds(o, _LANES)
            x = xs[sl]
            y = ys[sl]
            z = zs[sl]
            for m in range(_M):
                r = cf[m]
                dsts[3 * m][sl] = x * r[0] + y * r[1] + z * r[2] + r[9]
                dsts[3 * m + 1][sl] = x * r[3] + y * r[4] + z * r[5] + r[10]
                dsts[3 * m + 2][sl] = x * r[6] + y * r[7] + z * r[8] + r[11]

    for d in range(3):
        in_copy(0, d).start()
    for k in range(_NCHUNKS):
        slot = k % 2
        if k + 1 < _NCHUNKS:
            for d in range(3):
                in_copy(k + 1, d).start()
        for d in range(3):
            in_copy(k, d).wait()
        if k >= 2:
            for m in range(_M):
                for d in range(3):
                    out_copy(k - 2, m, d).wait()
        compute(slot)
        for m in range(_M):
            for d in range(3):
                out_copy(k, m, d).start()
    for k in (_NCHUNKS - 2, _NCHUNKS - 1):
        for m in range(_M):
            for d in range(3):
                out_copy(k, m, d).wait()


@jax.jit
def kernel(trans, pos):
    m, b = trans.shape[0], trans.shape[1]
    n = pos.shape[1]
    t = trans.reshape(m * b, 4, 4)
    rot = t[:, 0:3, 0:3].reshape(m * b, 9)
    shift = t[:, 0:3, 3]
    coef = jnp.concatenate(
        [rot, shift, jnp.zeros((m * b, 4), jnp.float32)], axis=1
    )
    coef = coef.reshape(m, b, 16).transpose(1, 0, 2).reshape(b, m * 16)
    xyz = jnp.transpose(pos, (2, 0, 1))

    mesh = plsc.VectorSubcoreMesh(
        core_axis_name="c", subcore_axis_name="s",
        num_cores=_NUM_CORES, num_subcores=_NUM_SUBCORES,
    )
    out = pl.kernel(
        _se3_body,
        out_type=jax.ShapeDtypeStruct((3, m * b, n), jnp.float32),
        mesh=mesh,
        scratch_types=(
            [
                [[pltpu.VMEM((_CHUNK,), jnp.float32) for _ in range(3)]
                 for _ in range(2)],
                [[pltpu.VMEM((_CHUNK,), jnp.float32) for _ in range(3 * _M)]
                 for _ in range(2)],
                pltpu.VMEM((_M * 16,), jnp.float32),
                [pltpu.SemaphoreType.DMA for _ in range(2)],
                [pltpu.SemaphoreType.DMA for _ in range(2)],
            ]
        ),
        compiler_params=pltpu.CompilerParams(needs_layout_passes=False),
    )(xyz, coef)
    return jnp.transpose(out, (1, 2, 0))
